# baseline (speedup 1.0000x reference)
import functools

import jax
import jax.numpy as jnp
from jax import lax
from jax.experimental import pallas as pl
from jax.experimental.pallas import tpu as pltpu
from jax.experimental.pallas import tpu_sc as plsc

DIM = 64
BATCH = 16384
NODES = 1000000

_NUM_CORES = 2
_NUM_SUBCORES = 16
_NUM_WORKERS = _NUM_CORES * _NUM_SUBCORES
_WIN = 512
_WBITS = 9
_NWIN = (NODES + _WIN - 1) // _WIN
_LAST_START = (_NWIN - 1) * _WIN
_TAIL = NODES - _LAST_START
_T_MAX = (_NWIN + _NUM_WORKERS - 1) // _NUM_WORKERS
_TAIL_T = (_NWIN - 1) // _NUM_WORKERS
_IDX_CHUNK = 2048
_NCHUNK = BATCH // _IDX_CHUNK
_RING = 256
_OROWS = BATCH + _NUM_WORKERS


def _gather_scan_sc(table_t, data, tail):
    mesh = plsc.VectorSubcoreMesh(core_axis_name="c", subcore_axis_name="s")

    @functools.partial(
        pl.kernel,
        mesh=mesh,
        out_type=jax.ShapeDtypeStruct((_OROWS, DIM), jnp.float32),
        scratch_types=[
            pltpu.VMEM((_IDX_CHUNK,), jnp.int32),
            pltpu.VMEM((BATCH + 16,), jnp.int32),
            pltpu.VMEM((_RING,), jnp.int32),
            pltpu.VMEM((2, DIM, _WIN), jnp.float32),
            pltpu.VMEM((2, 16, DIM), jnp.float32),
            pltpu.VMEM((DIM * _TAIL,), jnp.float32),
            pltpu.SemaphoreType.DMA,
            pltpu.SemaphoreType.DMA,
            pltpu.SemaphoreType.DMA,
        ],
        compiler_params=pltpu.CompilerParams(needs_layout_passes=False),
    )
    def k(
        tab_hbm,
        idx_hbm,
        tail_hbm,
        out_hbm,
        idxc_vm,
        keys_vm,
        ring_vm,
        slab_v,
        stage_v,
        tail_vm,
        sem_slab,
        sem_s0,
        sem_s1,
    ):
        cid = lax.axis_index("c")
        sid = lax.axis_index("s")
        wid = sid * _NUM_CORES + cid
        iota16 = jax.lax.iota(jnp.int32, 16)
        trash = BATCH + wid

        pltpu.sync_copy(tail_hbm.at[...], tail_vm)

        def compact_chunk(ci, off):
            pltpu.sync_copy(
                idx_hbm.at[
                    pl.ds(
                        pl.multiple_of(ci * _IDX_CHUNK, _IDX_CHUNK),
                        _IDX_CHUNK,
                    )
                ],
                idxc_vm,
            )

            def compact(g, off):
                nv = idxc_vm[pl.ds(g * 16, 16)]
                v = lax.shift_right_logical(nv, _WBITS)
                hit = lax.bitwise_and(v, _NUM_WORKERS - 1) == wid
                t = lax.shift_right_logical(v, 5)
                p = lax.bitwise_and(nv, _WIN - 1)
                b = ci * _IDX_CHUNK + g * 16 + iota16
                key = lax.bitwise_or(
                    lax.bitwise_or(
                        lax.shift_left(t, 23), lax.shift_left(p, 14)
                    ),
                    b,
                )
                npop = plsc.all_reduce_population_count(hit)[0]

                @pl.when(npop > 0)
                def _():
                    pref = jnp.cumsum(hit.astype(jnp.int32))
                    pos = off + pref - 1
                    pos_safe = jnp.where(hit, pos, BATCH + iota16)
                    plsc.store_scatter(keys_vm, [pos_safe], key)

                return off + npop

            return lax.fori_loop(0, _IDX_CHUNK // 16, compact, off)

        off = lax.fori_loop(0, _NCHUNK, compact_chunk, jnp.int32(0))
        nq = lax.shift_right_logical(off + 15, 4)

        def process_group(rbase, rem, cur, slot, sem_s):
            kv = ring_vm[pl.ds(rbase, 16)]
            act = iota16 < rem
            p = lax.bitwise_and(lax.shift_right_logical(kv, 14), _WIN - 1)
            b = lax.bitwise_and(kv, 16383)
            b_safe = jnp.where(act, b, trash)
            cur16 = jnp.full((16,), cur, jnp.int32)
            slot16 = jnp.full((16,), slot, jnp.int32)
            for d in range(DIM):
                d16 = jnp.full((16,), d, jnp.int32)
                x = plsc.load_gather(slab_v, [cur16, d16, p])
                plsc.store_scatter(stage_v, [slot16, iota16, d16], x)
            for jj in range(16):
                pltpu.async_copy(
                    stage_v.at[slot, pl.ds(jj, 1)],
                    out_hbm.at[pl.ds(b_safe[jj], 1)],
                    sem_s,
                )

        def drain_group(slot, sem_s):
            for _d in range(16):
                pltpu.make_async_copy(
                    stage_v.at[slot, pl.ds(0, 1)],
                    out_hbm.at[pl.ds(0, 1)],
                    sem_s,
                ).wait()

        def dispatch(rbase, rem, cur, G):
            @pl.when(lax.bitwise_and(G, 1) == 0)
            def _():
                @pl.when(G >= 2)
                def _():
                    drain_group(0, sem_s0)

                process_group(rbase, rem, cur, 0, sem_s0)

            @pl.when(lax.bitwise_and(G, 1) == 1)
            def _():
                @pl.when(G >= 2)
                def _():
                    drain_group(1, sem_s1)

                process_group(rbase, rem, cur, 1, sem_s1)

        pltpu.async_copy(
            tab_hbm.at[:, pl.ds(pl.multiple_of(wid * _WIN, _WIN), _WIN)],
            slab_v.at[0],
            sem_slab,
        )

        def win_body(t, carry):
            cw0, G0 = carry
            v = wid + t * _NUM_WORKERS
            cur = lax.bitwise_and(t, 1)
            nxt = 1 - cur
            vn = v + _NUM_WORKERS

            @pl.when(vn < _NWIN - 1)
            def _():
                pltpu.async_copy(
                    tab_hbm.at[
                        :, pl.ds(pl.multiple_of(vn * _WIN, _WIN), _WIN)
                    ],
                    slab_v.at[nxt],
                    sem_slab,
                )

            @pl.when(v < _NWIN - 1)
            def _():
                pltpu.make_async_copy(
                    tab_hbm.at[:, pl.ds(0, _WIN)],
                    slab_v.at[cur],
                    sem_slab,
                ).wait()

            @pl.when(v == _NWIN - 1)
            def _():
                for d in range(DIM):
                    for c4 in range(_TAIL // 16):
                        slab_v[_TAIL_T & 1, d, pl.ds(c4 * 16, 16)] = (
                            tail_vm[pl.ds(d * _TAIL + c4 * 16, 16)]
                        )

            def select(q, sc):
                cw, cr, G = sc
                kv = keys_vm[pl.ds(q * 16, 16)]
                lanes = q * 16 + iota16
                m = jnp.logical_and(
                    lanes < off, lax.shift_right_logical(kv, 23) == t
                )
                npop = plsc.all_reduce_population_count(m)[0]

                @pl.when(npop > 0)
                def _():
                    pref = jnp.cumsum(m.astype(jnp.int32))
                    pos = lax.bitwise_and(cw + pref - 1, _RING - 1)
                    dead = lax.bitwise_and(cr - 16 + iota16, _RING - 1)
                    pos_safe = jnp.where(m, pos, dead)
                    plsc.store_scatter(ring_vm, [pos_safe], kv)

                cw = cw + npop

                def proc(i, st):
                    cr, G = st
                    dispatch(lax.bitwise_and(cr, _RING - 1), 16, cur, G)
                    return cr + 16, G + 1

                navail = lax.shift_right_logical(cw - cr, 4)
                cr, G = lax.fori_loop(0, navail, proc, (cr, G))
                return cw, cr, G

            cw, cr, G = lax.fori_loop(0, nq, select, (cw0, cw0, G0))

            rem = cw - cr

            @pl.when(rem > 0)
            def _():
                dispatch(lax.bitwise_and(cr, _RING - 1), rem, cur, G)

            G = jnp.where(rem > 0, G + 1, G)
            cw_al = lax.bitwise_and(cw + 15, ~jnp.int32(15))
            return cw_al, G

        cw, G = lax.fori_loop(
            0, _T_MAX, win_body, (jnp.int32(0), jnp.int32(0))
        )

        @pl.when(G >= 2)
        def _():
            drain_group(0, sem_s0)
            drain_group(1, sem_s1)

        @pl.when(G == 1)
        def _():
            drain_group(0, sem_s0)

    return k(table_t, data, tail)


def _norm_body(x_ref, o_ref):
    x = x_ref[...]
    ss = jnp.sum(x * x, axis=1, keepdims=True)
    y = x / jnp.maximum(jnp.sqrt(ss), 1e-12)
    o_ref[...] = y.T


def _normalize_tc(rows):
    grid = 16
    blk = BATCH // grid
    return pl.pallas_call(
        _norm_body,
        grid=(grid,),
        in_specs=[pl.BlockSpec((blk, DIM), lambda i: (i, 0))],
        out_specs=pl.BlockSpec((DIM, blk), lambda i: (0, i)),
        out_shape=jax.ShapeDtypeStruct((DIM, BATCH), jnp.float32),
    )(rows)


@jax.jit
def kernel(data, table):
    table_t = table.T
    tail = table_t[:, _LAST_START:].reshape(DIM * _TAIL)
    gathered = _gather_scan_sc(table_t, data, tail)
    out_t = _normalize_tc(gathered[:BATCH])
    return out_t.T


# speedup vs baseline: 1.0509x; 1.0509x over previous
import functools

import jax
import jax.numpy as jnp
from jax import lax
from jax.experimental import pallas as pl
from jax.experimental.pallas import tpu as pltpu
from jax.experimental.pallas import tpu_sc as plsc

DIM = 64
BATCH = 16384
NODES = 1000000

_NUM_CORES = 2
_NUM_SUBCORES = 16
_NUM_WORKERS = _NUM_CORES * _NUM_SUBCORES
_WIN = 512
_WBITS = 9
_NWIN = (NODES + _WIN - 1) // _WIN
_LAST_START = (_NWIN - 1) * _WIN
_TAIL = NODES - _LAST_START
_T_MAX = (_NWIN + _NUM_WORKERS - 1) // _NUM_WORKERS
_TAIL_T = (_NWIN - 1) // _NUM_WORKERS
_IDX_CHUNK = 2048
_NCHUNK = BATCH // _IDX_CHUNK
_RING = 256
_OROWS = BATCH + _NUM_WORKERS


def _gather_scan_sc(table_t, data, tail):
    mesh = plsc.VectorSubcoreMesh(core_axis_name="c", subcore_axis_name="s")

    @functools.partial(
        pl.kernel,
        mesh=mesh,
        out_type=jax.ShapeDtypeStruct((_OROWS, DIM), jnp.float32),
        scratch_types=[
            pltpu.VMEM((_IDX_CHUNK,), jnp.int32),
            pltpu.VMEM((BATCH + 16,), jnp.int32),
            pltpu.VMEM((_RING,), jnp.int32),
            pltpu.VMEM((2, DIM, _WIN), jnp.float32),
            pltpu.VMEM((2, 16, DIM), jnp.float32),
            pltpu.VMEM((DIM * _TAIL,), jnp.float32),
            pltpu.SemaphoreType.DMA,
            pltpu.SemaphoreType.DMA,
            pltpu.SemaphoreType.DMA,
        ],
        compiler_params=pltpu.CompilerParams(needs_layout_passes=False),
    )
    def k(
        tab_hbm,
        idx_hbm,
        tail_hbm,
        out_hbm,
        idxc_vm,
        keys_vm,
        ring_vm,
        slab_v,
        stage_v,
        tail_vm,
        sem_slab,
        sem_s0,
        sem_s1,
    ):
        cid = lax.axis_index("c")
        sid = lax.axis_index("s")
        wid = sid * _NUM_CORES + cid
        iota16 = jax.lax.iota(jnp.int32, 16)
        trash = BATCH + wid

        pltpu.sync_copy(tail_hbm.at[...], tail_vm)

        def compact_chunk(ci, off):
            pltpu.sync_copy(
                idx_hbm.at[
                    pl.ds(
                        pl.multiple_of(ci * _IDX_CHUNK, _IDX_CHUNK),
                        _IDX_CHUNK,
                    )
                ],
                idxc_vm,
            )

            def compact(g, off):
                nv = idxc_vm[pl.ds(g * 16, 16)]
                v = lax.shift_right_logical(nv, _WBITS)
                hit = lax.bitwise_and(v, _NUM_WORKERS - 1) == wid
                t = lax.shift_right_logical(v, 5)
                p = lax.bitwise_and(nv, _WIN - 1)
                b = ci * _IDX_CHUNK + g * 16 + iota16
                key = lax.bitwise_or(
                    lax.bitwise_or(
                        lax.shift_left(t, 23), lax.shift_left(p, 14)
                    ),
                    b,
                )
                pref = jnp.cumsum(hit.astype(jnp.int32))
                pos = off + pref - 1
                pos_safe = jnp.where(hit, pos, BATCH + iota16)
                plsc.store_scatter(keys_vm, [pos_safe], key)
                return off + pref[15]

            return lax.fori_loop(0, _IDX_CHUNK // 16, compact, off)

        off = lax.fori_loop(0, _NCHUNK, compact_chunk, jnp.int32(0))
        nq = lax.shift_right_logical(off + 15, 4)

        def process_group(rbase, rem, cur, slot, sem_s):
            kv = ring_vm[pl.ds(rbase, 16)]
            act = iota16 < rem
            p = lax.bitwise_and(lax.shift_right_logical(kv, 14), _WIN - 1)
            b = lax.bitwise_and(kv, 16383)
            b_safe = jnp.where(act, b, trash)
            cur16 = jnp.full((16,), cur, jnp.int32)
            slot16 = jnp.full((16,), slot, jnp.int32)
            for d in range(DIM):
                d16 = jnp.full((16,), d, jnp.int32)
                x = plsc.load_gather(slab_v, [cur16, d16, p])
                plsc.store_scatter(stage_v, [slot16, iota16, d16], x)
            for jj in range(16):
                pltpu.async_copy(
                    stage_v.at[slot, pl.ds(jj, 1)],
                    out_hbm.at[pl.ds(b_safe[jj], 1)],
                    sem_s,
                )

        def drain_group(slot, sem_s):
            for _d in range(16):
                pltpu.make_async_copy(
                    stage_v.at[slot, pl.ds(0, 1)],
                    out_hbm.at[pl.ds(0, 1)],
                    sem_s,
                ).wait()

        def dispatch(rbase, rem, cur, G):
            @pl.when(lax.bitwise_and(G, 1) == 0)
            def _():
                @pl.when(G >= 2)
                def _():
                    drain_group(0, sem_s0)

                process_group(rbase, rem, cur, 0, sem_s0)

            @pl.when(lax.bitwise_and(G, 1) == 1)
            def _():
                @pl.when(G >= 2)
                def _():
                    drain_group(1, sem_s1)

                process_group(rbase, rem, cur, 1, sem_s1)

        pltpu.async_copy(
            tab_hbm.at[:, pl.ds(pl.multiple_of(wid * _WIN, _WIN), _WIN)],
            slab_v.at[0],
            sem_slab,
        )

        def win_body(t, carry):
            cw0, G0 = carry
            v = wid + t * _NUM_WORKERS
            cur = lax.bitwise_and(t, 1)
            nxt = 1 - cur
            vn = v + _NUM_WORKERS

            @pl.when(vn < _NWIN - 1)
            def _():
                pltpu.async_copy(
                    tab_hbm.at[
                        :, pl.ds(pl.multiple_of(vn * _WIN, _WIN), _WIN)
                    ],
                    slab_v.at[nxt],
                    sem_slab,
                )

            @pl.when(v < _NWIN - 1)
            def _():
                pltpu.make_async_copy(
                    tab_hbm.at[:, pl.ds(0, _WIN)],
                    slab_v.at[cur],
                    sem_slab,
                ).wait()

            @pl.when(v == _NWIN - 1)
            def _():
                for d in range(DIM):
                    for c4 in range(_TAIL // 16):
                        slab_v[_TAIL_T & 1, d, pl.ds(c4 * 16, 16)] = (
                            tail_vm[pl.ds(d * _TAIL + c4 * 16, 16)]
                        )

            def select(q, sc):
                cw, cr, G = sc
                kv = keys_vm[pl.ds(q * 16, 16)]
                lanes = q * 16 + iota16
                m = jnp.logical_and(
                    lanes < off, lax.shift_right_logical(kv, 23) == t
                )
                pref = jnp.cumsum(m.astype(jnp.int32))
                pos = lax.bitwise_and(cw + pref - 1, _RING - 1)
                dead = lax.bitwise_and(cr - 16 + iota16, _RING - 1)
                pos_safe = jnp.where(m, pos, dead)
                plsc.store_scatter(ring_vm, [pos_safe], kv)
                cw = cw + pref[15]

                def proc(i, st):
                    cr, G = st
                    dispatch(lax.bitwise_and(cr, _RING - 1), 16, cur, G)
                    return cr + 16, G + 1

                navail = lax.shift_right_logical(cw - cr, 4)
                cr, G = lax.fori_loop(0, navail, proc, (cr, G))
                return cw, cr, G

            cw, cr, G = lax.fori_loop(0, nq, select, (cw0, cw0, G0))

            rem = cw - cr

            @pl.when(rem > 0)
            def _():
                dispatch(lax.bitwise_and(cr, _RING - 1), rem, cur, G)

            G = jnp.where(rem > 0, G + 1, G)
            cw_al = lax.bitwise_and(cw + 15, ~jnp.int32(15))
            return cw_al, G

        cw, G = lax.fori_loop(
            0, _T_MAX, win_body, (jnp.int32(0), jnp.int32(0))
        )

        @pl.when(G >= 2)
        def _():
            drain_group(0, sem_s0)
            drain_group(1, sem_s1)

        @pl.when(G == 1)
        def _():
            drain_group(0, sem_s0)

    return k(table_t, data, tail)


def _norm_body(x_ref, o_ref):
    x = x_ref[...]
    ss = jnp.sum(x * x, axis=1, keepdims=True)
    y = x / jnp.maximum(jnp.sqrt(ss), 1e-12)
    o_ref[...] = y.T


def _normalize_tc(rows):
    grid = 16
    blk = BATCH // grid
    return pl.pallas_call(
        _norm_body,
        grid=(grid,),
        in_specs=[pl.BlockSpec((blk, DIM), lambda i: (i, 0))],
        out_specs=pl.BlockSpec((DIM, blk), lambda i: (0, i)),
        out_shape=jax.ShapeDtypeStruct((DIM, BATCH), jnp.float32),
    )(rows)


@jax.jit
def kernel(data, table):
    table_t = table.T
    tail = table_t[:, _LAST_START:].reshape(DIM * _TAIL)
    gathered = _gather_scan_sc(table_t, data, tail)
    out_t = _normalize_tc(gathered[:BATCH])
    return out_t.T
